# baseline (speedup 1.0000x reference)
import functools

import jax
import jax.numpy as jnp
from jax import lax
from jax.experimental import pallas as pl
from jax.experimental.pallas import tpu as pltpu
from jax.experimental.pallas import tpu_sc as plsc

_DIM = 128
_H = 512
_FAN = 3 * _DIM


def _full(k):
    return jnp.full((16,), k, dtype=jnp.int32)


def kernel(discrete_data, continuous_data, cat_table, sub_table, ind_table,
           W1, b1, W2, b2):
    mesh = plsc.VectorSubcoreMesh(core_axis_name="c", subcore_axis_name="s")

    @functools.partial(
        pl.kernel,
        mesh=mesh,
        compiler_params=pltpu.CompilerParams(needs_layout_passes=False),
        out_type=jax.ShapeDtypeStruct((32, 8, 16), jnp.float32),
        scratch_types=[
            pltpu.VMEM((3,), jnp.int32),
            pltpu.VMEM((1, 128), jnp.float32),
            pltpu.VMEM((3, _DIM), jnp.float32),
            pltpu.VMEM((3, _DIM), jnp.float32),
            pltpu.VMEM((3, _DIM), jnp.float32),
            pltpu.VMEM((16, _FAN + 3), jnp.float32),
            pltpu.VMEM((32, 16), jnp.float32),
            pltpu.VMEM((2, 16), jnp.float32),
            pltpu.VMEM((16, _DIM), jnp.float32),
            pltpu.VMEM((1, 8, 16), jnp.float32),
            pltpu.SemaphoreType.DMA,
        ],
    )
    def sc_kernel(disc_h, cont_h, cat_h, sub_h, ind_h, w1_h, b1_h, w2_h,
                  out_h, idx_v, cont_v, rows_cat, rows_sub, rows_ind,
                  w1_slab, b1_v, h_loc, w2t_slab, o_parts, sem):
        cid = lax.axis_index("c")
        sid = lax.axis_index("s")
        w = sid * 2 + cid

        pltpu.sync_copy(disc_h, idx_v)
        pltpu.sync_copy(cont_h, cont_v)
        pltpu.async_copy(cat_h.at[idx_v], rows_cat, sem).wait()
        pltpu.async_copy(sub_h.at[idx_v], rows_sub, sem).wait()
        pltpu.async_copy(ind_h.at[idx_v], rows_ind, sem).wait()
        pltpu.sync_copy(w1_h.at[pl.ds(w * 16, 16)], w1_slab)
        pltpu.sync_copy(b1_h, b1_v)
        pltpu.sync_copy(w2_h.at[pl.ds(w * 16, 16)], w2t_slab)

        iota16 = lax.iota(jnp.int32, 16)

        def l1_loop(rows_ref, row_i, col_off, acc):
            def body(k, acc):
                xk = plsc.load_gather(rows_ref, [_full(row_i), _full(k)])
                c = plsc.load_gather(w1_slab, [iota16, _full(col_off + k)])
                return acc + xk * c
            return lax.fori_loop(0, _DIM, body, acc)

        acc = jnp.zeros((16,), jnp.float32)
        acc = l1_loop(rows_cat, 0, 0, acc)
        acc = l1_loop(rows_sub, 1, _DIM, acc)
        acc = l1_loop(rows_ind, 2, 2 * _DIM, acc)
        for i in range(3):
            ck = plsc.load_gather(cont_v, [_full(0), _full(8 + i)])
            acc = acc + ck * plsc.load_gather(w1_slab,
                                              [iota16, _full(_FAN + i)])
        h_loc[1] = jnp.maximum(acc + b1_v[w], 0.0)

        for v in range(8):
            oacc = jnp.zeros((16,), jnp.float32)
            for k in range(16):
                hk = plsc.load_gather(h_loc, [_full(1), _full(k)])
                wc = plsc.load_gather(w2t_slab,
                                      [_full(k), iota16 + 16 * v])
                oacc = oacc + hk * wc
            o_parts[0, v] = oacc
        pltpu.sync_copy(o_parts, out_h.at[pl.ds(w, 1)])

    cont_pad = jnp.pad(continuous_data.reshape(1, 3), ((0, 0), (8, 117)))
    parts = sc_kernel(discrete_data, cont_pad, cat_table, sub_table,
                      ind_table, W1, b1.reshape(32, 16), W2.T)
    return parts.sum(axis=0).reshape(_DIM) + b2
